# baseline (speedup 1.0000x reference)
import functools
import jax
import jax.numpy as jnp
from jax import lax
from jax.experimental import pallas as pl
from jax.experimental.pallas import tpu as pltpu
from jax.experimental.pallas import tpu_sc as plsc

_N = 8388608
_NW = 32
_PER_W = _N // _NW
_CH = 16384
_CHUNKS = _PER_W // _CH
_VPC = _CH // 16
_UNROLL = 8


def _sc_l1_body(pred_hbm, gt_hbm, outs_hbm, outc_hbm,
                pbuf, gbuf, mbuf, obuf, psem, gsem, msem, osem):
    c = lax.axis_index("c")
    s = lax.axis_index("s")
    wid = s * 2 + c
    base = wid * _PER_W

    def start(k, slot):
        off = base + k * _CH
        pltpu.make_async_copy(
            pred_hbm.at[pl.ds(off, _CH)], pbuf.at[slot], psem).start()
        pltpu.make_async_copy(
            gt_hbm.at[pl.ds(off, _CH)], gbuf.at[slot], gsem).start()
        pltpu.make_async_copy(
            gt_hbm.at[pl.ds(_N + off, _CH)], mbuf.at[slot], msem).start()

    def wait(k, slot):
        off = base + k * _CH
        pltpu.make_async_copy(
            pred_hbm.at[pl.ds(off, _CH)], pbuf.at[slot], psem).wait()
        pltpu.make_async_copy(
            gt_hbm.at[pl.ds(off, _CH)], gbuf.at[slot], gsem).wait()
        pltpu.make_async_copy(
            gt_hbm.at[pl.ds(_N + off, _CH)], mbuf.at[slot], msem).wait()

    start(0, 0)
    acc_s = jnp.zeros((16,), jnp.float32)
    acc_c = jnp.zeros((16,), jnp.float32)
    for k in range(_CHUNKS):
        slot = k % 2
        if k + 1 < _CHUNKS:
            start(k + 1, 1 - slot)
        wait(k, slot)

        def inner(j, accs, slot=slot):
            sa, ca = accs
            for l in range(_UNROLL):
                o = (j * _UNROLL + l) * 16
                p = pbuf[slot, pl.ds(o, 16)]
                g = gbuf[slot, pl.ds(o, 16)]
                m = mbuf[slot, pl.ds(o, 16)] > 0.0
                sa = sa + jnp.where(m, jnp.abs(p - g), 0.0)
                ca = ca + jnp.where(m, 1.0, 0.0)
            return (sa, ca)

        acc_s, acc_c = lax.fori_loop(0, _VPC // _UNROLL, inner, (acc_s, acc_c))

    obuf[pl.ds(0, 16)] = acc_s
    obuf[pl.ds(16, 16)] = acc_c
    pltpu.make_async_copy(obuf.at[pl.ds(0, 16)], outs_hbm.at[pl.ds(wid * 16, 16)], osem).start()
    pltpu.make_async_copy(obuf.at[pl.ds(0, 16)], outs_hbm.at[pl.ds(wid * 16, 16)], osem).wait()
    pltpu.make_async_copy(obuf.at[pl.ds(16, 16)], outc_hbm.at[pl.ds(wid * 16, 16)], osem).start()
    pltpu.make_async_copy(obuf.at[pl.ds(16, 16)], outc_hbm.at[pl.ds(wid * 16, 16)], osem).wait()


def sc_partials(pred_flat, gt_flat):
    mesh = plsc.VectorSubcoreMesh(core_axis_name="c", subcore_axis_name="s")
    k = functools.partial(
        pl.kernel,
        mesh=mesh,
        out_type=[
            jax.ShapeDtypeStruct((_NW * 16,), jnp.float32),
            jax.ShapeDtypeStruct((_NW * 16,), jnp.float32),
        ],
        scratch_types=[
            pltpu.VMEM((2, _CH), jnp.float32),
            pltpu.VMEM((2, _CH), jnp.float32),
            pltpu.VMEM((2, _CH), jnp.float32),
            pltpu.VMEM((32,), jnp.float32),
            pltpu.SemaphoreType.DMA,
            pltpu.SemaphoreType.DMA,
            pltpu.SemaphoreType.DMA,
            pltpu.SemaphoreType.DMA,
        ],
    )(_sc_l1_body)
    return k(pred_flat, gt_flat)


def kernel(pred, gt):
    pred_flat = pred.reshape(_N)
    gt_flat = gt.reshape(2 * _N)
    s, c = sc_partials(pred_flat, gt_flat)
    return jnp.sum(s) / jnp.sum(c)


# speedup vs baseline: 1.5653x; 1.5653x over previous
import functools
import jax
import jax.numpy as jnp
from jax import lax
from jax.experimental import pallas as pl
from jax.experimental.pallas import tpu as pltpu
from jax.experimental.pallas import tpu_sc as plsc

_LEAD = 512
_B = 32
_T = 352
_TC_GRID = _T // _B

_N = _LEAD * 16384
_NW = 32
_E0 = _T * 16384
_PER_W = (_N - _E0) // _NW
_CH = 16384
_CHUNKS = _PER_W // _CH
_VPC = _CH // 16
_UNROLL = 8


def _tc_body(pred_ref, gtd_ref, msk_ref, out_ref, sacc_ref, cacc_ref):
    i = pl.program_id(0)

    @pl.when(i == 0)
    def _init():
        sacc_ref[...] = jnp.zeros_like(sacc_ref)
        cacc_ref[...] = jnp.zeros_like(cacc_ref)

    p = pred_ref[...]
    g = gtd_ref[...]
    m = msk_ref[...] > 0.0
    diff = jnp.where(m, jnp.abs(p - g), 0.0)
    cnt = m.astype(jnp.float32)
    sacc_ref[...] += jnp.sum(diff.reshape(_B * 16, 8, 128), axis=0)
    cacc_ref[...] += jnp.sum(cnt.reshape(_B * 16, 8, 128), axis=0)

    @pl.when(i == _TC_GRID - 1)
    def _fin():
        out_ref[0, 0] = jnp.sum(sacc_ref[...])
        out_ref[1, 0] = jnp.sum(cacc_ref[...])


def _tc_partials(pred3, gt3):
    return pl.pallas_call(
        _tc_body,
        grid=(_TC_GRID,),
        in_specs=[
            pl.BlockSpec((_B, 128, 128), lambda i: (i, 0, 0)),
            pl.BlockSpec((_B, 128, 128), lambda i: (i, 0, 0)),
            pl.BlockSpec((_B, 128, 128), lambda i: (i + _LEAD // _B, 0, 0)),
        ],
        out_specs=pl.BlockSpec(memory_space=pltpu.SMEM),
        out_shape=jax.ShapeDtypeStruct((2, 1), jnp.float32),
        scratch_shapes=[
            pltpu.VMEM((8, 128), jnp.float32),
            pltpu.VMEM((8, 128), jnp.float32),
        ],
    )(pred3, gt3, gt3)


def _sc_body(pred_hbm, gt_hbm, outs_hbm, outc_hbm,
             pbuf, gbuf, mbuf, obuf, psem, gsem, msem, osem):
    c = lax.axis_index("c")
    s = lax.axis_index("s")
    wid = s * 2 + c
    base = _E0 + wid * _PER_W

    def start(k, slot):
        off = base + k * _CH
        pltpu.make_async_copy(
            pred_hbm.at[pl.ds(off, _CH)], pbuf.at[slot], psem).start()
        pltpu.make_async_copy(
            gt_hbm.at[pl.ds(off, _CH)], gbuf.at[slot], gsem).start()
        pltpu.make_async_copy(
            gt_hbm.at[pl.ds(_N + off, _CH)], mbuf.at[slot], msem).start()

    def wait(k, slot):
        off = base + k * _CH
        pltpu.make_async_copy(
            pred_hbm.at[pl.ds(off, _CH)], pbuf.at[slot], psem).wait()
        pltpu.make_async_copy(
            gt_hbm.at[pl.ds(off, _CH)], gbuf.at[slot], gsem).wait()
        pltpu.make_async_copy(
            gt_hbm.at[pl.ds(_N + off, _CH)], mbuf.at[slot], msem).wait()

    start(0, 0)
    acc_s = jnp.zeros((16,), jnp.float32)
    acc_c = jnp.zeros((16,), jnp.float32)
    for k in range(_CHUNKS):
        slot = k % 2
        if k + 1 < _CHUNKS:
            start(k + 1, 1 - slot)
        wait(k, slot)

        def inner(j, accs, slot=slot):
            sa, ca = accs
            for l in range(_UNROLL):
                o = (j * _UNROLL + l) * 16
                p = pbuf[slot, pl.ds(o, 16)]
                g = gbuf[slot, pl.ds(o, 16)]
                m = mbuf[slot, pl.ds(o, 16)] > 0.0
                sa = sa + jnp.where(m, jnp.abs(p - g), 0.0)
                ca = ca + jnp.where(m, 1.0, 0.0)
            return (sa, ca)

        acc_s, acc_c = lax.fori_loop(0, _VPC // _UNROLL, inner, (acc_s, acc_c))

    obuf[pl.ds(0, 16)] = acc_s
    obuf[pl.ds(16, 16)] = acc_c
    pltpu.make_async_copy(
        obuf.at[pl.ds(0, 16)], outs_hbm.at[pl.ds(wid * 16, 16)], osem).start()
    pltpu.make_async_copy(
        obuf.at[pl.ds(0, 16)], outs_hbm.at[pl.ds(wid * 16, 16)], osem).wait()
    pltpu.make_async_copy(
        obuf.at[pl.ds(16, 16)], outc_hbm.at[pl.ds(wid * 16, 16)], osem).start()
    pltpu.make_async_copy(
        obuf.at[pl.ds(16, 16)], outc_hbm.at[pl.ds(wid * 16, 16)], osem).wait()


def _sc_partials(pred_flat, gt_flat):
    mesh = plsc.VectorSubcoreMesh(core_axis_name="c", subcore_axis_name="s")
    k = functools.partial(
        pl.kernel,
        mesh=mesh,
        out_type=[
            jax.ShapeDtypeStruct((_NW * 16,), jnp.float32),
            jax.ShapeDtypeStruct((_NW * 16,), jnp.float32),
        ],
        scratch_types=[
            pltpu.VMEM((2, _CH), jnp.float32),
            pltpu.VMEM((2, _CH), jnp.float32),
            pltpu.VMEM((2, _CH), jnp.float32),
            pltpu.VMEM((32,), jnp.float32),
            pltpu.SemaphoreType.DMA,
            pltpu.SemaphoreType.DMA,
            pltpu.SemaphoreType.DMA,
            pltpu.SemaphoreType.DMA,
        ],
    )(_sc_body)
    return k(pred_flat, gt_flat)


def kernel(pred, gt):
    pred3 = pred.reshape(_LEAD, 128, 128)
    gt3 = gt.reshape(2 * _LEAD, 128, 128)
    sc_s, sc_c = _sc_partials(pred.reshape(_N), gt.reshape(2 * _N))
    tc = _tc_partials(pred3, gt3)
    total_s = tc[0, 0] + jnp.sum(sc_s)
    total_c = tc[1, 0] + jnp.sum(sc_c)
    return total_s / total_c


# speedup vs baseline: 1.5745x; 1.0059x over previous
import functools
import jax
import jax.numpy as jnp
from jax import lax
from jax.experimental import pallas as pl
from jax.experimental.pallas import tpu as pltpu
from jax.experimental.pallas import tpu_sc as plsc

_LEAD = 512
_B = 32
_T = 416
_TC_GRID = _T // _B

_N = _LEAD * 16384
_NW = 32
_E0 = _T * 16384
_PER_W = (_N - _E0) // _NW
_CH = 16384
_CHUNKS = _PER_W // _CH
_VPC = _CH // 16
_UNROLL = 8


def _tc_body(pred_ref, gtd_ref, msk_ref, out_ref, sacc_ref, cacc_ref):
    i = pl.program_id(0)

    @pl.when(i == 0)
    def _init():
        sacc_ref[...] = jnp.zeros_like(sacc_ref)
        cacc_ref[...] = jnp.zeros_like(cacc_ref)

    p = pred_ref[...]
    g = gtd_ref[...]
    m = msk_ref[...] > 0.0
    diff = jnp.where(m, jnp.abs(p - g), 0.0)
    cnt = m.astype(jnp.float32)
    sacc_ref[...] += jnp.sum(diff.reshape(_B * 16, 8, 128), axis=0)
    cacc_ref[...] += jnp.sum(cnt.reshape(_B * 16, 8, 128), axis=0)

    @pl.when(i == _TC_GRID - 1)
    def _fin():
        out_ref[0, 0] = jnp.sum(sacc_ref[...])
        out_ref[1, 0] = jnp.sum(cacc_ref[...])


def _tc_partials(pred3, gt3):
    return pl.pallas_call(
        _tc_body,
        grid=(_TC_GRID,),
        in_specs=[
            pl.BlockSpec((_B, 128, 128), lambda i: (i, 0, 0)),
            pl.BlockSpec((_B, 128, 128), lambda i: (i, 0, 0)),
            pl.BlockSpec((_B, 128, 128), lambda i: (i + _LEAD // _B, 0, 0)),
        ],
        out_specs=pl.BlockSpec(memory_space=pltpu.SMEM),
        out_shape=jax.ShapeDtypeStruct((2, 1), jnp.float32),
        scratch_shapes=[
            pltpu.VMEM((8, 128), jnp.float32),
            pltpu.VMEM((8, 128), jnp.float32),
        ],
    )(pred3, gt3, gt3)


def _sc_body(pred_hbm, gt_hbm, outs_hbm, outc_hbm,
             pbuf, gbuf, mbuf, obuf, psem, gsem, msem, osem):
    c = lax.axis_index("c")
    s = lax.axis_index("s")
    wid = s * 2 + c
    base = _E0 + wid * _PER_W

    def start(k, slot):
        off = base + k * _CH
        pltpu.make_async_copy(
            pred_hbm.at[pl.ds(off, _CH)], pbuf.at[slot], psem).start()
        pltpu.make_async_copy(
            gt_hbm.at[pl.ds(off, _CH)], gbuf.at[slot], gsem).start()
        pltpu.make_async_copy(
            gt_hbm.at[pl.ds(_N + off, _CH)], mbuf.at[slot], msem).start()

    def wait(k, slot):
        off = base + k * _CH
        pltpu.make_async_copy(
            pred_hbm.at[pl.ds(off, _CH)], pbuf.at[slot], psem).wait()
        pltpu.make_async_copy(
            gt_hbm.at[pl.ds(off, _CH)], gbuf.at[slot], gsem).wait()
        pltpu.make_async_copy(
            gt_hbm.at[pl.ds(_N + off, _CH)], mbuf.at[slot], msem).wait()

    start(0, 0)
    acc_s = jnp.zeros((16,), jnp.float32)
    acc_c = jnp.zeros((16,), jnp.float32)
    for k in range(_CHUNKS):
        slot = k % 2
        if k + 1 < _CHUNKS:
            start(k + 1, 1 - slot)
        wait(k, slot)

        def inner(j, accs, slot=slot):
            sa, ca = accs
            for l in range(_UNROLL):
                o = (j * _UNROLL + l) * 16
                p = pbuf[slot, pl.ds(o, 16)]
                g = gbuf[slot, pl.ds(o, 16)]
                m = mbuf[slot, pl.ds(o, 16)] > 0.0
                sa = sa + jnp.where(m, jnp.abs(p - g), 0.0)
                ca = ca + jnp.where(m, 1.0, 0.0)
            return (sa, ca)

        acc_s, acc_c = lax.fori_loop(0, _VPC // _UNROLL, inner, (acc_s, acc_c))

    obuf[pl.ds(0, 16)] = acc_s
    obuf[pl.ds(16, 16)] = acc_c
    pltpu.make_async_copy(
        obuf.at[pl.ds(0, 16)], outs_hbm.at[pl.ds(wid * 16, 16)], osem).start()
    pltpu.make_async_copy(
        obuf.at[pl.ds(0, 16)], outs_hbm.at[pl.ds(wid * 16, 16)], osem).wait()
    pltpu.make_async_copy(
        obuf.at[pl.ds(16, 16)], outc_hbm.at[pl.ds(wid * 16, 16)], osem).start()
    pltpu.make_async_copy(
        obuf.at[pl.ds(16, 16)], outc_hbm.at[pl.ds(wid * 16, 16)], osem).wait()


def _sc_partials(pred_flat, gt_flat):
    mesh = plsc.VectorSubcoreMesh(core_axis_name="c", subcore_axis_name="s")
    k = functools.partial(
        pl.kernel,
        mesh=mesh,
        out_type=[
            jax.ShapeDtypeStruct((_NW * 16,), jnp.float32),
            jax.ShapeDtypeStruct((_NW * 16,), jnp.float32),
        ],
        scratch_types=[
            pltpu.VMEM((2, _CH), jnp.float32),
            pltpu.VMEM((2, _CH), jnp.float32),
            pltpu.VMEM((2, _CH), jnp.float32),
            pltpu.VMEM((32,), jnp.float32),
            pltpu.SemaphoreType.DMA,
            pltpu.SemaphoreType.DMA,
            pltpu.SemaphoreType.DMA,
            pltpu.SemaphoreType.DMA,
        ],
    )(_sc_body)
    return k(pred_flat, gt_flat)


def kernel(pred, gt):
    pred3 = pred.reshape(_LEAD, 128, 128)
    gt3 = gt.reshape(2 * _LEAD, 128, 128)
    sc_s, sc_c = _sc_partials(pred.reshape(_N), gt.reshape(2 * _N))
    tc = _tc_partials(pred3, gt3)
    total_s = tc[0, 0] + jnp.sum(sc_s)
    total_c = tc[1, 0] + jnp.sum(sc_c)
    return total_s / total_c


# speedup vs baseline: 1.6170x; 1.0270x over previous
import functools
import jax
import jax.numpy as jnp
from jax import lax
from jax.experimental import pallas as pl
from jax.experimental.pallas import tpu as pltpu
from jax.experimental.pallas import tpu_sc as plsc

_LEAD = 512
_B = 32
_T = 416
_TC_GRID = _T // _B

_N = _LEAD * 16384
_NW = 32
_E0 = _T * 16384
_PER_W = (_N - _E0) // _NW
_CH = 16384
_CHUNKS = _PER_W // _CH
_VPC = _CH // 16
_UNROLL = 8


def _tc_body(pred_ref, gtd_ref, msk_ref, out_ref, sacc_ref, cacc_ref):
    i = pl.program_id(0)

    @pl.when(i == 0)
    def _init():
        sacc_ref[...] = jnp.zeros_like(sacc_ref)
        cacc_ref[...] = jnp.zeros_like(cacc_ref)

    p = pred_ref[...]
    g = gtd_ref[...]
    m = msk_ref[...] > 0.0
    diff = jnp.where(m, jnp.abs(p - g), 0.0)
    cnt = m.astype(jnp.float32)
    sacc_ref[...] += jnp.sum(diff.reshape(_B * 16, 8, 128), axis=0)
    cacc_ref[...] += jnp.sum(cnt.reshape(_B * 16, 8, 128), axis=0)

    @pl.when(i == _TC_GRID - 1)
    def _fin():
        out_ref[0, 0] = jnp.sum(sacc_ref[...])
        out_ref[1, 0] = jnp.sum(cacc_ref[...])


def _tc_partials(pred3, gt3):
    return pl.pallas_call(
        _tc_body,
        grid=(_TC_GRID,),
        in_specs=[
            pl.BlockSpec((_B, 128, 128), lambda i: (i, 0, 0)),
            pl.BlockSpec((_B, 128, 128), lambda i: (i, 0, 0)),
            pl.BlockSpec((_B, 128, 128), lambda i: (i + _LEAD // _B, 0, 0)),
        ],
        out_specs=pl.BlockSpec(memory_space=pltpu.SMEM),
        out_shape=jax.ShapeDtypeStruct((2, 1), jnp.float32),
        scratch_shapes=[
            pltpu.VMEM((8, 128), jnp.float32),
            pltpu.VMEM((8, 128), jnp.float32),
        ],
    )(pred3, gt3, gt3)


def _sc_body(pred_hbm, gt_hbm, out_hbm,
             pbuf, gbuf, mbuf, obuf, psem, gsem, msem, osem):
    c = lax.axis_index("c")
    s = lax.axis_index("s")
    wid = s * 2 + c
    base = _E0 + wid * _PER_W

    def start(k, slot):
        off = base + k * _CH
        pltpu.make_async_copy(
            pred_hbm.at[pl.ds(off, _CH)], pbuf.at[slot], psem).start()
        pltpu.make_async_copy(
            gt_hbm.at[pl.ds(off, _CH)], gbuf.at[slot], gsem).start()
        pltpu.make_async_copy(
            gt_hbm.at[pl.ds(_N + off, _CH)], mbuf.at[slot], msem).start()

    def wait(k, slot):
        off = base + k * _CH
        pltpu.make_async_copy(
            pred_hbm.at[pl.ds(off, _CH)], pbuf.at[slot], psem).wait()
        pltpu.make_async_copy(
            gt_hbm.at[pl.ds(off, _CH)], gbuf.at[slot], gsem).wait()
        pltpu.make_async_copy(
            gt_hbm.at[pl.ds(_N + off, _CH)], mbuf.at[slot], msem).wait()

    start(0, 0)
    acc_s = jnp.zeros((16,), jnp.float32)
    acc_c = jnp.zeros((16,), jnp.float32)
    for k in range(_CHUNKS):
        slot = k % 2
        if k + 1 < _CHUNKS:
            start(k + 1, 1 - slot)
        wait(k, slot)

        def inner(j, accs, slot=slot):
            sa, ca = accs
            for l in range(_UNROLL):
                o = (j * _UNROLL + l) * 16
                p = pbuf[slot, pl.ds(o, 16)]
                g = gbuf[slot, pl.ds(o, 16)]
                m = mbuf[slot, pl.ds(o, 16)] > 0.0
                sa = sa + jnp.where(m, jnp.abs(p - g), 0.0)
                ca = ca + jnp.where(m, 1.0, 0.0)
            return (sa, ca)

        acc_s, acc_c = lax.fori_loop(0, _VPC // _UNROLL, inner, (acc_s, acc_c))

    obuf[pl.ds(0, 16)] = acc_s
    obuf[pl.ds(16, 16)] = acc_c
    pltpu.make_async_copy(
        obuf.at[pl.ds(0, 32)], out_hbm.at[pl.ds(wid * 32, 32)], osem).start()
    pltpu.make_async_copy(
        obuf.at[pl.ds(0, 32)], out_hbm.at[pl.ds(wid * 32, 32)], osem).wait()


def _sc_partials(pred_flat, gt_flat):
    mesh = plsc.VectorSubcoreMesh(core_axis_name="c", subcore_axis_name="s")
    k = functools.partial(
        pl.kernel,
        mesh=mesh,
        out_type=jax.ShapeDtypeStruct((_NW * 32,), jnp.float32),
        scratch_types=[
            pltpu.VMEM((2, _CH), jnp.float32),
            pltpu.VMEM((2, _CH), jnp.float32),
            pltpu.VMEM((2, _CH), jnp.float32),
            pltpu.VMEM((32,), jnp.float32),
            pltpu.SemaphoreType.DMA,
            pltpu.SemaphoreType.DMA,
            pltpu.SemaphoreType.DMA,
            pltpu.SemaphoreType.DMA,
        ],
    )(_sc_body)
    return k(pred_flat, gt_flat)


def kernel(pred, gt):
    pred3 = pred.reshape(_LEAD, 128, 128)
    gt3 = gt.reshape(2 * _LEAD, 128, 128)
    sc_sc = _sc_partials(pred.reshape(_N), gt.reshape(2 * _N))
    sc2 = sc_sc.reshape(_NW, 2, 16)
    tc = _tc_partials(pred3, gt3)
    total_s = tc[0, 0] + jnp.sum(sc2[:, 0, :])
    total_c = tc[1, 0] + jnp.sum(sc2[:, 1, :])
    return total_s / total_c


# speedup vs baseline: 2.7109x; 1.6765x over previous
import jax
import jax.numpy as jnp
from jax.experimental import pallas as pl
from jax.experimental.pallas import tpu as pltpu

_LEAD = 512
_B = 64
_GRID = _LEAD // _B


def _l1_body(pred_ref, gtd_ref, msk_ref, out_ref, sacc_ref, cacc_ref):
    i = pl.program_id(0)

    @pl.when(i == 0)
    def _init():
        sacc_ref[...] = jnp.zeros_like(sacc_ref)
        cacc_ref[...] = jnp.zeros_like(cacc_ref)

    p = pred_ref[...]
    g = gtd_ref[...]
    m = msk_ref[...] > 0.0
    diff = jnp.where(m, jnp.abs(p - g), 0.0)
    cnt = m.astype(jnp.float32)
    sacc_ref[...] += jnp.sum(diff.reshape(_B * 16, 8, 128), axis=0)
    cacc_ref[...] += jnp.sum(cnt.reshape(_B * 16, 8, 128), axis=0)

    @pl.when(i == _GRID - 1)
    def _fin():
        out_ref[0, 0] = jnp.sum(sacc_ref[...]) / jnp.sum(cacc_ref[...])


def kernel(pred, gt):
    pred3 = pred.reshape(_LEAD, 128, 128)
    gt3 = gt.reshape(2 * _LEAD, 128, 128)
    out = pl.pallas_call(
        _l1_body,
        grid=(_GRID,),
        in_specs=[
            pl.BlockSpec((_B, 128, 128), lambda i: (i, 0, 0)),
            pl.BlockSpec((_B, 128, 128), lambda i: (i, 0, 0)),
            pl.BlockSpec((_B, 128, 128), lambda i: (i + _GRID, 0, 0)),
        ],
        out_specs=pl.BlockSpec(memory_space=pltpu.SMEM),
        out_shape=jax.ShapeDtypeStruct((1, 1), jnp.float32),
        scratch_shapes=[
            pltpu.VMEM((8, 128), jnp.float32),
            pltpu.VMEM((8, 128), jnp.float32),
        ],
    )(pred3, gt3, gt3)
    return out[0, 0]


# speedup vs baseline: 2.7402x; 1.0108x over previous
import jax
import jax.numpy as jnp
from jax.experimental import pallas as pl
from jax.experimental.pallas import tpu as pltpu

_LEAD = 512
_B = 32
_GRID = _LEAD // _B


def _l1_body(pred_ref, gtd_ref, msk_ref, out_ref, sacc_ref, cacc_ref):
    i = pl.program_id(0)

    @pl.when(i == 0)
    def _init():
        sacc_ref[...] = jnp.zeros_like(sacc_ref)
        cacc_ref[...] = jnp.zeros_like(cacc_ref)

    p = pred_ref[...]
    g = gtd_ref[...]
    m = msk_ref[...] > 0.0
    diff = jnp.where(m, jnp.abs(p - g), 0.0)
    cnt = m.astype(jnp.float32)
    sacc_ref[...] += jnp.sum(diff.reshape(_B * 16, 8, 128), axis=0)
    cacc_ref[...] += jnp.sum(cnt.reshape(_B * 16, 8, 128), axis=0)

    @pl.when(i == _GRID - 1)
    def _fin():
        out_ref[0, 0] = jnp.sum(sacc_ref[...]) / jnp.sum(cacc_ref[...])


def kernel(pred, gt):
    pred3 = pred.reshape(_LEAD, 128, 128)
    gt3 = gt.reshape(2 * _LEAD, 128, 128)
    out = pl.pallas_call(
        _l1_body,
        grid=(_GRID,),
        in_specs=[
            pl.BlockSpec((_B, 128, 128), lambda i: (i, 0, 0)),
            pl.BlockSpec((_B, 128, 128), lambda i: (i, 0, 0)),
            pl.BlockSpec((_B, 128, 128), lambda i: (i + _GRID, 0, 0)),
        ],
        out_specs=pl.BlockSpec(memory_space=pltpu.SMEM),
        out_shape=jax.ShapeDtypeStruct((1, 1), jnp.float32),
        scratch_shapes=[
            pltpu.VMEM((8, 128), jnp.float32),
            pltpu.VMEM((8, 128), jnp.float32),
        ],
    )(pred3, gt3, gt3)
    return out[0, 0]
